# Initial kernel scaffold; baseline (speedup 1.0000x reference)
import jax
import jax.numpy as jnp
from jax.experimental import pallas as pl


def kernel(edge_index, x, weight, bias):
    raise NotImplementedError("write your pallas kernel here")


# speedup vs baseline: 6.1180x; 6.1180x over previous
import functools

import jax
import jax.numpy as jnp
from jax.experimental import pallas as pl
from jax.experimental.pallas import tpu as pltpu


def _ru(x, m):
    return ((x + m - 1) // m) * m


def _proj_kernel(x_ref, w_ref, d_ref, z_ref):
    xb = x_ref[...].astype(jnp.bfloat16)
    z = jnp.dot(xb, w_ref[...], preferred_element_type=jnp.float32)
    z_ref[...] = (z * d_ref[...]).astype(z_ref.dtype)


def _prop1_kernel(a_ref, z_ref, d2_ref, h_ref):
    acc = jnp.dot(a_ref[...], z_ref[...], preferred_element_type=jnp.float32)
    h_ref[...] = (acc * d2_ref[...]).astype(h_ref.dtype)


def _prop2_kernel(a_ref, h_ref, d_ref, b_ref, o_ref, *, num_classes, c_pad):
    acc = jnp.dot(a_ref[...], h_ref[...], preferred_element_type=jnp.float32)
    logits = acc * d_ref[...] + b_ref[...]
    if num_classes < c_pad:
        col = jax.lax.broadcasted_iota(jnp.int32, logits.shape, 1)
        valid = col < num_classes
        logits = jnp.where(valid, logits, jnp.float32(-1e30))
        m = jnp.max(logits, axis=1, keepdims=True)
        e = jnp.where(valid, jnp.exp(logits - m), jnp.float32(0.0))
    else:
        m = jnp.max(logits, axis=1, keepdims=True)
        e = jnp.exp(logits - m)
    lse = jnp.log(jnp.sum(e, axis=1, keepdims=True)) + m
    o_ref[...] = logits - lse


def kernel(edge_index, x, weight, bias):
    N, F = x.shape
    C = weight.shape[1]
    tm = 512
    n_pad = _ru(N, tm)
    f_pad = _ru(F, 128)
    c_pad = _ru(C, 128)
    grid_rows = n_pad // tm

    src, dst = edge_index[0], edge_index[1]
    rows = jnp.concatenate([src, dst])
    cols = jnp.concatenate([dst, src])
    a = jnp.zeros((n_pad, n_pad), jnp.bfloat16)
    a = a.at[rows, cols].set(jnp.bfloat16(1.0))
    diag = jnp.arange(N)
    a = a.at[diag, diag].add(jnp.bfloat16(1.0))

    deg = jnp.sum(a, axis=1, dtype=jnp.float32)
    dinv = jnp.where(deg > 0, jax.lax.rsqrt(deg), jnp.float32(0.0))
    d1 = jnp.broadcast_to(dinv[:, None], (n_pad, c_pad))
    d2 = jnp.broadcast_to((dinv * dinv)[:, None], (n_pad, c_pad))

    if N == n_pad and F == f_pad:
        x_p = x
    else:
        x_p = jnp.zeros((n_pad, f_pad), x.dtype).at[:N, :F].set(x)
    w_p = jnp.zeros((f_pad, c_pad), jnp.bfloat16).at[:F, :C].set(
        weight.astype(jnp.bfloat16))
    b_p = jnp.zeros((1, c_pad), jnp.float32).at[0, :C].set(
        bias.astype(jnp.float32))

    params = pltpu.CompilerParams(
        dimension_semantics=("parallel",),
        vmem_limit_bytes=60 * 1024 * 1024)

    z0 = pl.pallas_call(
        _proj_kernel,
        out_shape=jax.ShapeDtypeStruct((n_pad, c_pad), jnp.bfloat16),
        grid=(grid_rows,),
        in_specs=[
            pl.BlockSpec((tm, f_pad), lambda i: (i, 0)),
            pl.BlockSpec((f_pad, c_pad), lambda i: (0, 0)),
            pl.BlockSpec((tm, c_pad), lambda i: (i, 0)),
        ],
        out_specs=pl.BlockSpec((tm, c_pad), lambda i: (i, 0)),
        compiler_params=params,
    )(x_p, w_p, d1)

    h = pl.pallas_call(
        _prop1_kernel,
        out_shape=jax.ShapeDtypeStruct((n_pad, c_pad), jnp.bfloat16),
        grid=(grid_rows,),
        in_specs=[
            pl.BlockSpec((tm, n_pad), lambda i: (i, 0)),
            pl.BlockSpec((n_pad, c_pad), lambda i: (0, 0)),
            pl.BlockSpec((tm, c_pad), lambda i: (i, 0)),
        ],
        out_specs=pl.BlockSpec((tm, c_pad), lambda i: (i, 0)),
        compiler_params=params,
    )(a, z0, d2)

    out_p = pl.pallas_call(
        functools.partial(_prop2_kernel, num_classes=C, c_pad=c_pad),
        out_shape=jax.ShapeDtypeStruct((n_pad, c_pad), jnp.float32),
        grid=(grid_rows,),
        in_specs=[
            pl.BlockSpec((tm, n_pad), lambda i: (i, 0)),
            pl.BlockSpec((n_pad, c_pad), lambda i: (0, 0)),
            pl.BlockSpec((tm, c_pad), lambda i: (i, 0)),
            pl.BlockSpec((1, c_pad), lambda i: (0, 0)),
        ],
        out_specs=pl.BlockSpec((tm, c_pad), lambda i: (i, 0)),
        compiler_params=params,
    )(a, h, d1, b_p)

    return out_p[:N, :C]


# speedup vs baseline: 9.1513x; 1.4958x over previous
import functools

import jax
import jax.numpy as jnp
from jax.experimental import pallas as pl
from jax.experimental.pallas import tpu as pltpu


def _ru(x, m):
    return ((x + m - 1) // m) * m


def _proj_kernel(x_ref, w_ref, d_ref, z_ref):
    xb = x_ref[...].astype(jnp.bfloat16)
    z = jnp.dot(xb, w_ref[...], preferred_element_type=jnp.float32)
    z_ref[...] = (z * d_ref[...]).astype(z_ref.dtype)


def _prop1_kernel(a_ref, z_ref, d2_ref, h_ref, *, tm):
    i = pl.program_id(0)
    acc = jnp.dot(a_ref[...], z_ref[...], preferred_element_type=jnp.float32)
    acc += z_ref[pl.ds(i * tm, tm), :].astype(jnp.float32)
    h_ref[...] = (acc * d2_ref[...]).astype(h_ref.dtype)


def _prop2_kernel(a_ref, h_ref, d_ref, b_ref, o_ref, *, num_classes, c_pad, tm):
    i = pl.program_id(0)
    acc = jnp.dot(a_ref[...], h_ref[...], preferred_element_type=jnp.float32)
    acc += h_ref[pl.ds(i * tm, tm), :].astype(jnp.float32)
    logits = acc * d_ref[...] + b_ref[...]
    if num_classes < c_pad:
        col = jax.lax.broadcasted_iota(jnp.int32, logits.shape, 1)
        valid = col < num_classes
        logits = jnp.where(valid, logits, jnp.float32(-1e30))
        m = jnp.max(logits, axis=1, keepdims=True)
        e = jnp.where(valid, jnp.exp(logits - m), jnp.float32(0.0))
    else:
        m = jnp.max(logits, axis=1, keepdims=True)
        e = jnp.exp(logits - m)
    lse = jnp.log(jnp.sum(e, axis=1, keepdims=True)) + m
    o_ref[...] = logits - lse


def kernel(edge_index, x, weight, bias):
    N, F = x.shape
    C = weight.shape[1]
    tm = 512
    n_pad = _ru(N, tm)
    f_pad = _ru(F, 128)
    c_pad = _ru(C, 128)
    grid_rows = n_pad // tm

    src, dst = edge_index[0], edge_index[1]
    rows = jnp.concatenate([src, dst])
    cols = jnp.concatenate([dst, src])
    a = jnp.zeros((n_pad, n_pad), jnp.bfloat16)
    a = a.at[rows, cols].set(jnp.bfloat16(1.0))

    deg = jnp.sum(a, axis=1, dtype=jnp.float32)
    deg = deg + (jnp.arange(n_pad) < N).astype(jnp.float32)
    dinv = jnp.where(deg > 0, jax.lax.rsqrt(deg), jnp.float32(0.0))
    d1 = jnp.broadcast_to(dinv[:, None], (n_pad, c_pad))
    d2 = jnp.broadcast_to((dinv * dinv)[:, None], (n_pad, c_pad))

    if N == n_pad and F == f_pad:
        x_p = x
    else:
        x_p = jnp.zeros((n_pad, f_pad), x.dtype).at[:N, :F].set(x)
    w_p = jnp.zeros((f_pad, c_pad), jnp.bfloat16).at[:F, :C].set(
        weight.astype(jnp.bfloat16))
    b_p = jnp.zeros((1, c_pad), jnp.float32).at[0, :C].set(
        bias.astype(jnp.float32))

    params = pltpu.CompilerParams(
        dimension_semantics=("parallel",),
        vmem_limit_bytes=60 * 1024 * 1024)

    z0 = pl.pallas_call(
        _proj_kernel,
        out_shape=jax.ShapeDtypeStruct((n_pad, c_pad), jnp.bfloat16),
        grid=(grid_rows,),
        in_specs=[
            pl.BlockSpec((tm, f_pad), lambda i: (i, 0)),
            pl.BlockSpec((f_pad, c_pad), lambda i: (0, 0)),
            pl.BlockSpec((tm, c_pad), lambda i: (i, 0)),
        ],
        out_specs=pl.BlockSpec((tm, c_pad), lambda i: (i, 0)),
        compiler_params=params,
    )(x_p, w_p, d1)

    h = pl.pallas_call(
        functools.partial(_prop1_kernel, tm=tm),
        out_shape=jax.ShapeDtypeStruct((n_pad, c_pad), jnp.bfloat16),
        grid=(grid_rows,),
        in_specs=[
            pl.BlockSpec((tm, n_pad), lambda i: (i, 0)),
            pl.BlockSpec((n_pad, c_pad), lambda i: (0, 0)),
            pl.BlockSpec((tm, c_pad), lambda i: (i, 0)),
        ],
        out_specs=pl.BlockSpec((tm, c_pad), lambda i: (i, 0)),
        compiler_params=params,
    )(a, z0, d2)

    out_p = pl.pallas_call(
        functools.partial(_prop2_kernel, num_classes=C, c_pad=c_pad, tm=tm),
        out_shape=jax.ShapeDtypeStruct((n_pad, c_pad), jnp.float32),
        grid=(grid_rows,),
        in_specs=[
            pl.BlockSpec((tm, n_pad), lambda i: (i, 0)),
            pl.BlockSpec((n_pad, c_pad), lambda i: (0, 0)),
            pl.BlockSpec((tm, c_pad), lambda i: (i, 0)),
            pl.BlockSpec((1, c_pad), lambda i: (0, 0)),
        ],
        out_specs=pl.BlockSpec((tm, c_pad), lambda i: (i, 0)),
        compiler_params=params,
    )(a, h, d1, b_p)

    return out_p[:N, :C]


# speedup vs baseline: 10.4140x; 1.1380x over previous
import functools

import jax
import jax.numpy as jnp
from jax.experimental import pallas as pl
from jax.experimental.pallas import tpu as pltpu


def _ru(x, m):
    return ((x + m - 1) // m) * m


def _proj_kernel(x_ref, w_ref, d_ref, z_ref):
    xb = x_ref[...].astype(jnp.bfloat16)
    z = jnp.dot(xb, w_ref[...], preferred_element_type=jnp.float32)
    z_ref[...] = (z * d_ref[...]).astype(z_ref.dtype)


def _prop1_kernel(a_ref, z_ref, d2_ref, h_ref, *, tm):
    i = pl.program_id(0)
    ab = a_ref[...].astype(jnp.bfloat16)
    acc = jnp.dot(ab, z_ref[...], preferred_element_type=jnp.float32)
    acc += z_ref[pl.ds(i * tm, tm), :].astype(jnp.float32)
    h_ref[...] = (acc * d2_ref[...]).astype(h_ref.dtype)


def _prop2_kernel(a_ref, h_ref, d_ref, b_ref, o_ref, *, num_classes, c_pad, tm):
    i = pl.program_id(0)
    ab = a_ref[...].astype(jnp.bfloat16)
    acc = jnp.dot(ab, h_ref[...], preferred_element_type=jnp.float32)
    acc += h_ref[pl.ds(i * tm, tm), :].astype(jnp.float32)
    logits = acc * d_ref[...] + b_ref[...]
    if num_classes < c_pad:
        col = jax.lax.broadcasted_iota(jnp.int32, logits.shape, 1)
        valid = col < num_classes
        logits = jnp.where(valid, logits, jnp.float32(-1e30))
        m = jnp.max(logits, axis=1, keepdims=True)
        e = jnp.where(valid, jnp.exp(logits - m), jnp.float32(0.0))
    else:
        m = jnp.max(logits, axis=1, keepdims=True)
        e = jnp.exp(logits - m)
    lse = jnp.log(jnp.sum(e, axis=1, keepdims=True)) + m
    o_ref[...] = logits - lse


def kernel(edge_index, x, weight, bias):
    N, F = x.shape
    C = weight.shape[1]
    tm = 512
    n_pad = _ru(N, tm)
    f_pad = _ru(F, 128)
    c_pad = _ru(C, 128)
    grid_rows = n_pad // tm

    src, dst = edge_index[0], edge_index[1]
    rows = jnp.concatenate([src, dst])
    cols = jnp.concatenate([dst, src])
    a = jnp.zeros((n_pad, n_pad), jnp.int8)
    a = a.at[rows, cols].set(jnp.int8(1))

    deg = jnp.sum(a, axis=1, dtype=jnp.int32).astype(jnp.float32)
    deg = deg + (jnp.arange(n_pad) < N).astype(jnp.float32)
    dinv = jnp.where(deg > 0, jax.lax.rsqrt(deg), jnp.float32(0.0))
    d1 = jnp.broadcast_to(dinv[:, None], (n_pad, c_pad))
    d2 = jnp.broadcast_to((dinv * dinv)[:, None], (n_pad, c_pad))

    if N == n_pad and F == f_pad:
        x_p = x
    else:
        x_p = jnp.zeros((n_pad, f_pad), x.dtype).at[:N, :F].set(x)
    w_p = jnp.zeros((f_pad, c_pad), jnp.bfloat16).at[:F, :C].set(
        weight.astype(jnp.bfloat16))
    b_p = jnp.zeros((1, c_pad), jnp.float32).at[0, :C].set(
        bias.astype(jnp.float32))

    params = pltpu.CompilerParams(
        dimension_semantics=("parallel",),
        vmem_limit_bytes=60 * 1024 * 1024)

    z0 = pl.pallas_call(
        _proj_kernel,
        out_shape=jax.ShapeDtypeStruct((n_pad, c_pad), jnp.bfloat16),
        grid=(grid_rows,),
        in_specs=[
            pl.BlockSpec((tm, f_pad), lambda i: (i, 0)),
            pl.BlockSpec((f_pad, c_pad), lambda i: (0, 0)),
            pl.BlockSpec((tm, c_pad), lambda i: (i, 0)),
        ],
        out_specs=pl.BlockSpec((tm, c_pad), lambda i: (i, 0)),
        compiler_params=params,
    )(x_p, w_p, d1)

    h = pl.pallas_call(
        functools.partial(_prop1_kernel, tm=tm),
        out_shape=jax.ShapeDtypeStruct((n_pad, c_pad), jnp.bfloat16),
        grid=(grid_rows,),
        in_specs=[
            pl.BlockSpec((tm, n_pad), lambda i: (i, 0)),
            pl.BlockSpec((n_pad, c_pad), lambda i: (0, 0)),
            pl.BlockSpec((tm, c_pad), lambda i: (i, 0)),
        ],
        out_specs=pl.BlockSpec((tm, c_pad), lambda i: (i, 0)),
        compiler_params=params,
    )(a, z0, d2)

    out_p = pl.pallas_call(
        functools.partial(_prop2_kernel, num_classes=C, c_pad=c_pad, tm=tm),
        out_shape=jax.ShapeDtypeStruct((n_pad, c_pad), jnp.float32),
        grid=(grid_rows,),
        in_specs=[
            pl.BlockSpec((tm, n_pad), lambda i: (i, 0)),
            pl.BlockSpec((n_pad, c_pad), lambda i: (0, 0)),
            pl.BlockSpec((tm, c_pad), lambda i: (i, 0)),
            pl.BlockSpec((1, c_pad), lambda i: (0, 0)),
        ],
        out_specs=pl.BlockSpec((tm, c_pad), lambda i: (i, 0)),
        compiler_params=params,
    )(a, h, d1, b_p)

    return out_p[:N, :C]
